# Initial kernel scaffold; baseline (speedup 1.0000x reference)
import jax
import jax.numpy as jnp
from jax.experimental import pallas as pl


def kernel(pc, intensity, sn, label, node_a, node_b, params):
    raise NotImplementedError("write your pallas kernel here")


# speedup vs baseline: 28.1994x; 28.1994x over previous
import functools

import jax
import jax.numpy as jnp
from jax import lax
from jax.experimental import pallas as pl
from jax.experimental.pallas import tpu as pltpu
from jax.experimental.pallas import tpu_sc as plsc

B, N, MA, MB = 2, 20480, 128, 128
K_AB = 32
TN = 512
NT = N // TN
MCH = 32
MC = MB // MCH

_HI = jax.lax.Precision.HIGHEST


def _relu(x):
    return jnp.maximum(x, 0.0)


def _p1_body(pcp_ref, na_ref, min3_ref, cm_ref, sums_ref):
    nt = pl.program_id(1)
    pcp = pcp_ref[0]
    na = na_ref[0]
    d2 = None
    for c in range(3):
        t = pcp[:, c:c + 1] - na[c:c + 1, :]
        t2 = t * t
        d2 = t2 if d2 is None else d2 + t2
    d = jnp.sqrt(d2)
    lanes = lax.broadcasted_iota(jnp.int32, (TN, MA), 1)
    ik0 = None
    for k in range(3):
        mn = jnp.min(d, axis=1, keepdims=True)
        ik = jnp.min(jnp.where(d == mn, lanes, MA), axis=1, keepdims=True)
        min3_ref[0, :, k:k + 1] = ik
        if k == 0:
            ik0 = ik
        if k < 2:
            d = jnp.where(lanes == ik, jnp.float32(jnp.inf), d)
    mask = (lanes == ik0).astype(jnp.float32)
    part = lax.dot_general(pcp, mask, (((0,), (0,)), ((), ())),
                           preferred_element_type=jnp.float32,
                           precision=_HI)

    @pl.when(nt == 0)
    def _():
        sums_ref[...] = jnp.zeros_like(sums_ref)

    sums_ref[...] += part

    @pl.when(nt == NT - 1)
    def _():
        s = sums_ref[...]
        cm_ref[0] = s[:3] / (s[3:4] + 1e-5)


def _pass1(pcp, node_a):
    return pl.pallas_call(
        _p1_body,
        grid=(B, NT),
        in_specs=[
            pl.BlockSpec((1, TN, 4), lambda b, nt: (b, nt, 0)),
            pl.BlockSpec((1, 3, MA), lambda b, nt: (b, 0, 0)),
        ],
        out_specs=[
            pl.BlockSpec((1, TN, 3), lambda b, nt: (b, nt, 0)),
            pl.BlockSpec((1, 3, MA), lambda b, nt: (b, 0, 0)),
        ],
        out_shape=[
            jax.ShapeDtypeStruct((B, N, 3), jnp.int32),
            jax.ShapeDtypeStruct((B, 3, MA), jnp.float32),
        ],
        scratch_shapes=[pltpu.VMEM((4, MA), jnp.float32)],
    )(pcp, node_a)


def _p2_body(aug0_ref, mi_ref, cmp_ref, w1, b1, w2, b2, w3, b3,
             ct_ref, first_ref):
    ik = mi_ref[0]
    lanes = lax.broadcasted_iota(jnp.int32, (TN, MA), 1)
    mask = (lanes == ik).astype(jnp.float32)
    centers8 = jnp.dot(mask, cmp_ref[0], precision=_HI)
    ct_ref[0] = centers8[:, :3]
    x = aug0_ref[0] - centers8
    x = _relu(jnp.dot(x, w1[...]) + b1[...])
    x = _relu(jnp.dot(x, w2[...]) + b2[...])
    x = _relu(jnp.dot(x, w3[...]) + b3[...])
    first_ref[0] = x


def _pass2(aug0, mi, cmp, fw, fb):
    wspecs = []
    for w in (fw[0], fb[0], fw[1], fb[1], fw[2], fb[2]):
        wspecs.append(pl.BlockSpec(w.shape, lambda b, nt: (0,) * w.ndim))
    return pl.pallas_call(
        _p2_body,
        grid=(B, NT),
        in_specs=[
            pl.BlockSpec((1, TN, 8), lambda b, nt: (b, nt, 0)),
            pl.BlockSpec((1, TN, 1), lambda b, nt: (b, nt, 0)),
            pl.BlockSpec((1, MA, 8), lambda b, nt: (b, 0, 0)),
        ] + wspecs,
        out_specs=[
            pl.BlockSpec((1, TN, 3), lambda b, nt: (b, nt, 0)),
            pl.BlockSpec((1, TN, 32), lambda b, nt: (b, nt, 0)),
        ],
        out_shape=[
            jax.ShapeDtypeStruct((B, N, 3), jnp.float32),
            jax.ShapeDtypeStruct((B, N, 32), jnp.float32),
        ],
    )(aug0, mi, cmp, fw[0], fb[0], fw[1], fb[1], fw[2], fb[2])


def _segmax_sc(feat, idx, C):
    NC, NS = 2, 16
    NW = NC * NS
    n_per_w = feat.shape[1]
    G = C // 16
    mesh = plsc.VectorSubcoreMesh(core_axis_name="c", subcore_axis_name="s",
                                  num_cores=NC, num_subcores=NS)

    feat = feat.reshape(NW, n_per_w * C)

    @functools.partial(
        pl.kernel,
        out_type=jax.ShapeDtypeStruct((NW, MA * C), jnp.float32),
        mesh=mesh,
        scratch_types=[
            pltpu.VMEM((n_per_w * C,), jnp.float32),
            pltpu.VMEM((n_per_w,), jnp.int32),
            pltpu.VMEM((MA * C,), jnp.float32),
        ],
    )
    def k(feat_hbm, idx_hbm, out_hbm, feat_v, idx_v, acc_v):
        w = lax.axis_index("s") * NC + lax.axis_index("c")
        pltpu.sync_copy(feat_hbm.at[w], feat_v)
        pltpu.sync_copy(idx_hbm.at[w], idx_v)

        def zero(i, carry):
            acc_v[pl.ds(i * 16, 16)] = jnp.zeros((16,), jnp.float32)
            return carry

        lax.fori_loop(0, MA * C // 16, zero, 0)

        def body(i, carry):
            iv = idx_v[pl.ds(i * 16, 16)]
            for j in range(16):
                base = iv[j] * C
                fbase = (i * 16 + j) * C
                for g in range(G):
                    v = feat_v[pl.ds(fbase + g * 16, 16)]
                    a = acc_v[pl.ds(base + g * 16, 16)]
                    acc_v[pl.ds(base + g * 16, 16)] = jnp.maximum(a, v)
            return carry

        lax.fori_loop(0, n_per_w // 16, body, 0)
        pltpu.sync_copy(acc_v, out_hbm.at[w])

    return k(feat, idx)


def _segmax(feat_bnc, idx_bn, C):
    NW = 32
    n_per_w = (B * N) // NW
    f = feat_bnc.reshape(NW, n_per_w, C)
    i = idx_bn.reshape(NW, n_per_w)
    part = _segmax_sc(f, i, C)
    return part.reshape(B, NW // B, MA, C)


def _p3_body(first_ref, mi_ref, fmp_ref, w1a, w1b, b1, w2, b2, second_ref):
    fm = fmp_ref[0, 0]
    for t in range(1, fmp_ref.shape[1]):
        fm = jnp.maximum(fm, fmp_ref[0, t])
    ik = mi_ref[0]
    lanes = lax.broadcasted_iota(jnp.int32, (TN, MA), 1)
    mask = (lanes == ik).astype(jnp.float32)
    sc = jnp.dot(mask, fm, precision=_HI)
    f = first_ref[0]
    h = _relu(jnp.dot(f, w1a[...]) + jnp.dot(sc, w1b[...]) + b1[...])
    second_ref[0] = _relu(jnp.dot(h, w2[...]) + b2[...])


def _pass3(first, mi, fmp, w1a, w1b, b1, w2, b2):
    wspecs = [pl.BlockSpec(w.shape, lambda b, nt: (0,) * w.ndim)
              for w in (w1a, w1b, b1, w2, b2)]
    nw_b = fmp.shape[1]
    return pl.pallas_call(
        _p3_body,
        grid=(B, NT),
        in_specs=[
            pl.BlockSpec((1, TN, 32), lambda b, nt: (b, nt, 0)),
            pl.BlockSpec((1, TN, 1), lambda b, nt: (b, nt, 0)),
            pl.BlockSpec((1, nw_b, MA, 32), lambda b, nt: (b, 0, 0, 0)),
        ] + wspecs,
        out_specs=[pl.BlockSpec((1, TN, 64), lambda b, nt: (b, nt, 0))],
        out_shape=[jax.ShapeDtypeStruct((B, N, 64), jnp.float32)],
    )(first, mi, fmp, w1a, w1b, b1, w2, b2)[0]


def _b1_body(cm_ref, nbt_ref, cmt_ref, nafp_ref,
             k1wa, k1wb, k1b1, k1w2, k1b2,
             k2wa, k2wb, k2b1, k2w2, k2b2,
             nbf_ref, naf_ref):
    mc = pl.program_id(1)
    naf = nafp_ref[0, 0]
    for t in range(1, nafp_ref.shape[1]):
        naf = jnp.maximum(naf, nafp_ref[0, t])

    @pl.when(mc == 0)
    def _():
        naf_ref[0] = naf

    cm = cm_ref[0]
    nbc = nbt_ref[0]
    d2 = None
    for c in range(3):
        t = nbc[:, c:c + 1] - cm[c:c + 1, :]
        t2 = t * t
        d2 = t2 if d2 is None else d2 + t2
    d = jnp.sqrt(d2 + 1e-12)
    lanes = lax.broadcasted_iota(jnp.int32, (MCH, MA), 1)
    cols = []
    for k in range(K_AB):
        mn = jnp.min(d, axis=1, keepdims=True)
        ik = jnp.min(jnp.where(d == mn, lanes, MA), axis=1, keepdims=True)
        cols.append(ik)
        if k < K_AB - 1:
            d = jnp.where(lanes == ik, jnp.float32(jnp.inf), d)
    idxcol = jnp.concatenate(cols, axis=0)
    R = K_AB * MCH
    lanesr = lax.broadcasted_iota(jnp.int32, (R, MA), 1)
    O = (lanesr == idxcol).astype(jnp.float32)
    nb_coord = jnp.dot(O, cmt_ref[0], precision=_HI)
    nbb = jnp.concatenate([nbc] * K_AB, axis=0)
    rel = nb_coord - nbb
    nb_feat = jnp.dot(O, naf, precision=_HI)
    x1 = _relu(jnp.dot(rel, k1wa[...]) + jnp.dot(nb_feat, k1wb[...])
               + k1b1[...])
    x1 = _relu(jnp.dot(x1, k1w2[...]) + k1b2[...])
    pooled = x1[0:MCH]
    for k in range(1, K_AB):
        pooled = jnp.maximum(pooled, x1[k * MCH:(k + 1) * MCH])
    ph = jnp.dot(pooled, k2wa[...])
    phr = jnp.concatenate([ph] * K_AB, axis=0)
    x2 = _relu(phr + jnp.dot(x1, k2wb[...]) + k2b1[...])
    x2 = _relu(jnp.dot(x2, k2w2[...]) + k2b2[...])
    nbf = x2[0:MCH]
    for k in range(1, K_AB):
        nbf = jnp.maximum(nbf, x2[k * MCH:(k + 1) * MCH])
    nbf_ref[0] = nbf


def _partb1(cm, nbt, cmt, nafp, ws):
    wspecs = [pl.BlockSpec(w.shape, lambda b, mc: (0,) * w.ndim) for w in ws]
    nw_b = nafp.shape[1]
    return pl.pallas_call(
        _b1_body,
        grid=(B, MC),
        in_specs=[
            pl.BlockSpec((1, 3, MA), lambda b, mc: (b, 0, 0)),
            pl.BlockSpec((1, MCH, 3), lambda b, mc: (b, mc, 0)),
            pl.BlockSpec((1, MA, 3), lambda b, mc: (b, 0, 0)),
            pl.BlockSpec((1, nw_b, MA, 64), lambda b, mc: (b, 0, 0, 0)),
        ] + wspecs,
        out_specs=[
            pl.BlockSpec((1, MCH, 256), lambda b, mc: (b, mc, 0)),
            pl.BlockSpec((1, MA, 64), lambda b, mc: (b, 0, 0)),
        ],
        out_shape=[
            jax.ShapeDtypeStruct((B, MB, 256), jnp.float32),
            jax.ShapeDtypeStruct((B, MA, 64), jnp.float32),
        ],
    )(cm, nbt, cmt, nafp, *ws)


def _b2_body(nbt_ref, nbf_ref, w1a, w1b, b1, w2, b2, g_ref):
    x = _relu(jnp.dot(nbt_ref[0], w1a[...]) + jnp.dot(nbf_ref[0], w1b[...])
              + b1[...])
    x = _relu(jnp.dot(x, w2[...]) + b2[...])
    g_ref[0] = jnp.max(x, axis=0, keepdims=True)


def _partb2(nbt, nbf, ws):
    wspecs = [pl.BlockSpec(w.shape, lambda b: (0,) * w.ndim) for w in ws]
    return pl.pallas_call(
        _b2_body,
        grid=(B,),
        in_specs=[
            pl.BlockSpec((1, MB, 3), lambda b: (b, 0, 0)),
            pl.BlockSpec((1, MB, 256), lambda b: (b, 0, 0)),
        ] + wspecs,
        out_specs=[pl.BlockSpec((1, 1, 512), lambda b: (b, 0, 0))],
        out_shape=[jax.ShapeDtypeStruct((B, 1, 512), jnp.float32)],
    )(nbt, nbf, *ws)[0]


def kernel(pc, intensity, sn, label, node_a, node_b, params):
    p = params
    pct = pc.transpose(0, 2, 1)
    pcp = jnp.concatenate(
        [pct, jnp.ones((B, N, 1), jnp.float32)], axis=2)

    min3, cm = _pass1(pcp, node_a)
    mi = min3[:, :, :1]

    cmt = cm.transpose(0, 2, 1)
    cmp = jnp.concatenate(
        [cmt, jnp.zeros((B, MA, 5), jnp.float32)], axis=2)
    aug0 = jnp.concatenate(
        [pct, intensity.transpose(0, 2, 1), sn.transpose(0, 2, 1),
         label.transpose(0, 2, 1)], axis=2)

    fw = [w.T for w in p['fp_W']]
    fb = [b.reshape(1, -1) for b in p['fp_b']]
    ct, first = _pass2(aug0, mi, cmp, fw, fb)

    idx_bn = min3[:, :, 0]
    fmp = _segmax(first, idx_bn, 32)

    sw1 = p['sp_W'][0].T
    second = _pass3(first, mi, fmp, sw1[:32], sw1[32:],
                    p['sp_b'][0].reshape(1, -1),
                    p['sp_W'][1].T, p['sp_b'][1].reshape(1, -1))

    nafp = _segmax(second, idx_bn, 64)

    nbt = node_b.transpose(0, 2, 1)
    k1w1 = p['k1_W'][0].T
    k2w1 = p['k2_W'][0].T
    bws = [k1w1[:3], k1w1[3:], p['k1_b'][0].reshape(1, -1),
           p['k1_W'][1].T, p['k1_b'][1].reshape(1, -1),
           k2w1[:256], k2w1[256:], p['k2_b'][0].reshape(1, -1),
           p['k2_W'][1].T, p['k2_b'][1].reshape(1, -1)]
    nbf, naf = _partb1(cm, nbt, cmt, nafp, bws)

    fiw1 = p['fi_W'][0].T
    g = _partb2(nbt, nbf, [fiw1[:3], fiw1[3:], p['fi_b'][0].reshape(1, -1),
                           p['fi_W'][1].T, p['fi_b'][1].reshape(1, -1)])

    return (ct.transpose(0, 2, 1),
            cm,
            min3,
            first.transpose(0, 2, 1),
            second.transpose(0, 2, 1),
            naf.transpose(0, 2, 1),
            nbf.transpose(0, 2, 1),
            g.transpose(0, 2, 1))


# speedup vs baseline: 44.2712x; 1.5699x over previous
import functools

import jax
import jax.numpy as jnp
from jax import lax
from jax.experimental import pallas as pl
from jax.experimental.pallas import tpu as pltpu
from jax.experimental.pallas import tpu_sc as plsc

B, N, MA, MB = 2, 20480, 128, 128
K_AB = 32
TN = 2048
NT = N // TN
MCH = 64
MC = MB // MCH

_HI = jax.lax.Precision.HIGHEST


def _relu(x):
    return jnp.maximum(x, 0.0)


def _p1_body(pcp_ref, na_ref, min3_ref, cm_ref, sums_ref):
    nt = pl.program_id(1)
    pcp = pcp_ref[0]
    na = na_ref[0]
    d2 = None
    for c in range(3):
        t = pcp[:, c:c + 1] - na[c:c + 1, :]
        t2 = t * t
        d2 = t2 if d2 is None else d2 + t2
    d = jnp.sqrt(d2)
    lanes = lax.broadcasted_iota(jnp.int32, (TN, MA), 1).astype(jnp.float32)
    ik0 = None
    for k in range(3):
        mn = jnp.min(d, axis=1, keepdims=True)
        ik = jnp.min(jnp.where(d == mn, lanes, jnp.float32(MA)),
                     axis=1, keepdims=True)
        min3_ref[0, :, k:k + 1] = ik.astype(jnp.int32)
        if k == 0:
            ik0 = ik
        if k < 2:
            d = jnp.where(lanes == ik, jnp.float32(jnp.inf), d)
    mask = (lanes == ik0).astype(jnp.float32)
    part = lax.dot_general(pcp, mask, (((0,), (0,)), ((), ())),
                           preferred_element_type=jnp.float32,
                           precision=_HI)

    @pl.when(nt == 0)
    def _():
        sums_ref[...] = jnp.zeros_like(sums_ref)

    sums_ref[...] += part

    @pl.when(nt == NT - 1)
    def _():
        s = sums_ref[...]
        cm_ref[0] = s[:3] / (s[3:4] + 1e-5)


def _pass1(pcp, node_a):
    return pl.pallas_call(
        _p1_body,
        grid=(B, NT),
        in_specs=[
            pl.BlockSpec((1, TN, 4), lambda b, nt: (b, nt, 0)),
            pl.BlockSpec((1, 3, MA), lambda b, nt: (b, 0, 0)),
        ],
        out_specs=[
            pl.BlockSpec((1, TN, 3), lambda b, nt: (b, nt, 0)),
            pl.BlockSpec((1, 3, MA), lambda b, nt: (b, 0, 0)),
        ],
        out_shape=[
            jax.ShapeDtypeStruct((B, N, 3), jnp.int32),
            jax.ShapeDtypeStruct((B, 3, MA), jnp.float32),
        ],
        scratch_shapes=[pltpu.VMEM((4, MA), jnp.float32)],
    )(pcp, node_a)


def _p2_body(aug0_ref, mi_ref, cmp_ref, w1, b1, w2, b2, w3, b3,
             ct_ref, first_ref):
    ik = mi_ref[0]
    lanes = lax.broadcasted_iota(jnp.int32, (TN, MA), 1)
    mask = (lanes == ik).astype(jnp.float32)
    centers8 = jnp.dot(mask, cmp_ref[0])
    ct_ref[0] = centers8[:, :3]
    x = aug0_ref[0] - centers8
    x = _relu(jnp.dot(x, w1[...]) + b1[...])
    x = _relu(jnp.dot(x, w2[...]) + b2[...])
    x = _relu(jnp.dot(x, w3[...]) + b3[...])
    first_ref[0] = x


def _pass2(aug0, mi, cmp, fw, fb):
    wspecs = []
    for w in (fw[0], fb[0], fw[1], fb[1], fw[2], fb[2]):
        wspecs.append(pl.BlockSpec(w.shape, lambda b, nt: (0,) * w.ndim))
    return pl.pallas_call(
        _p2_body,
        grid=(B, NT),
        in_specs=[
            pl.BlockSpec((1, TN, 8), lambda b, nt: (b, nt, 0)),
            pl.BlockSpec((1, TN, 1), lambda b, nt: (b, nt, 0)),
            pl.BlockSpec((1, MA, 8), lambda b, nt: (b, 0, 0)),
        ] + wspecs,
        out_specs=[
            pl.BlockSpec((1, TN, 3), lambda b, nt: (b, nt, 0)),
            pl.BlockSpec((1, TN, 32), lambda b, nt: (b, nt, 0)),
        ],
        out_shape=[
            jax.ShapeDtypeStruct((B, N, 3), jnp.float32),
            jax.ShapeDtypeStruct((B, N, 32), jnp.float32),
        ],
    )(aug0, mi, cmp, fw[0], fb[0], fw[1], fb[1], fw[2], fb[2])


def _segmax_sc(feat, idx, C):
    NC, NS = 2, 16
    NW = NC * NS
    n_per_w = feat.shape[1]
    G = C // 16
    mesh = plsc.VectorSubcoreMesh(core_axis_name="c", subcore_axis_name="s",
                                  num_cores=NC, num_subcores=NS)

    feat = feat.reshape(NW, n_per_w * C)

    @functools.partial(
        pl.kernel,
        out_type=jax.ShapeDtypeStruct((NW, MA * C), jnp.float32),
        mesh=mesh,
        scratch_types=[
            pltpu.VMEM((n_per_w * C,), jnp.float32),
            pltpu.VMEM((n_per_w,), jnp.int32),
            pltpu.VMEM((MA * C,), jnp.float32),
        ],
    )
    def k(feat_hbm, idx_hbm, out_hbm, feat_v, idx_v, acc_v):
        w = lax.axis_index("s") * NC + lax.axis_index("c")
        pltpu.sync_copy(feat_hbm.at[w], feat_v)
        pltpu.sync_copy(idx_hbm.at[w], idx_v)

        def zero(i, carry):
            acc_v[pl.ds(i * 16, 16)] = jnp.zeros((16,), jnp.float32)
            return carry

        lax.fori_loop(0, MA * C // 16, zero, 0)

        def body(i, carry):
            iv = idx_v[pl.ds(i * 16, 16)]
            for j in range(16):
                base = iv[j] * C
                fbase = (i * 16 + j) * C
                for g in range(G):
                    v = feat_v[pl.ds(fbase + g * 16, 16)]
                    a = acc_v[pl.ds(base + g * 16, 16)]
                    acc_v[pl.ds(base + g * 16, 16)] = jnp.maximum(a, v)
            return carry

        lax.fori_loop(0, n_per_w // 16, body, 0)
        pltpu.sync_copy(acc_v, out_hbm.at[w])

    return k(feat, idx)


def _segmax(feat_bnc, idx_bn, C):
    NW = 32
    n_per_w = (B * N) // NW
    f = feat_bnc.reshape(NW, n_per_w, C)
    i = idx_bn.reshape(NW, n_per_w)
    part = _segmax_sc(f, i, C)
    return part.reshape(B, NW // B, MA, C)


def _p3_body(first_ref, mi_ref, fmp_ref, w1a, w1b, b1, w2, b2, second_ref):
    fm = fmp_ref[0, 0]
    for t in range(1, fmp_ref.shape[1]):
        fm = jnp.maximum(fm, fmp_ref[0, t])
    ik = mi_ref[0]
    lanes = lax.broadcasted_iota(jnp.int32, (TN, MA), 1)
    mask = (lanes == ik).astype(jnp.float32)
    sc = jnp.dot(mask, fm)
    f = first_ref[0]
    h = _relu(jnp.dot(f, w1a[...]) + jnp.dot(sc, w1b[...]) + b1[...])
    second_ref[0] = _relu(jnp.dot(h, w2[...]) + b2[...])


def _pass3(first, mi, fmp, w1a, w1b, b1, w2, b2):
    wspecs = [pl.BlockSpec(w.shape, lambda b, nt: (0,) * w.ndim)
              for w in (w1a, w1b, b1, w2, b2)]
    nw_b = fmp.shape[1]
    return pl.pallas_call(
        _p3_body,
        grid=(B, NT),
        in_specs=[
            pl.BlockSpec((1, TN, 32), lambda b, nt: (b, nt, 0)),
            pl.BlockSpec((1, TN, 1), lambda b, nt: (b, nt, 0)),
            pl.BlockSpec((1, nw_b, MA, 32), lambda b, nt: (b, 0, 0, 0)),
        ] + wspecs,
        out_specs=[pl.BlockSpec((1, TN, 64), lambda b, nt: (b, nt, 0))],
        out_shape=[jax.ShapeDtypeStruct((B, N, 64), jnp.float32)],
    )(first, mi, fmp, w1a, w1b, b1, w2, b2)[0]


def _b1_body(cm_ref, nbt_ref, cmt_ref, nafp_ref,
             k1wa, k1wb, k1b1, k1w2, k1b2,
             k2wa, k2wb, k2b1, k2w2, k2b2,
             nbf_ref, naf_ref):
    mc = pl.program_id(1)
    naf = nafp_ref[0, 0]
    for t in range(1, nafp_ref.shape[1]):
        naf = jnp.maximum(naf, nafp_ref[0, t])

    @pl.when(mc == 0)
    def _():
        naf_ref[0] = naf

    cm = cm_ref[0]
    nbc = nbt_ref[0]
    d2 = None
    for c in range(3):
        t = nbc[:, c:c + 1] - cm[c:c + 1, :]
        t2 = t * t
        d2 = t2 if d2 is None else d2 + t2
    d = jnp.sqrt(d2 + 1e-12)
    lanes = lax.broadcasted_iota(jnp.int32, (MCH, MA), 1).astype(jnp.float32)
    cols = []
    for k in range(K_AB):
        mn = jnp.min(d, axis=1, keepdims=True)
        ik = jnp.min(jnp.where(d == mn, lanes, jnp.float32(MA)),
                     axis=1, keepdims=True)
        cols.append(ik)
        if k < K_AB - 1:
            d = jnp.where(lanes == ik, jnp.float32(jnp.inf), d)
    idxcol = jnp.concatenate(cols, axis=0)
    R = K_AB * MCH
    lanesr = lax.broadcasted_iota(jnp.int32, (R, MA), 1).astype(jnp.float32)
    O = (lanesr == idxcol).astype(jnp.float32)
    nb_coord = jnp.dot(O, cmt_ref[0])
    nbb = jnp.concatenate([nbc] * K_AB, axis=0)
    rel = nb_coord - nbb
    nb_feat = jnp.dot(O, naf)
    x1 = _relu(jnp.dot(rel, k1wa[...]) + jnp.dot(nb_feat, k1wb[...])
               + k1b1[...])
    x1 = _relu(jnp.dot(x1, k1w2[...]) + k1b2[...])
    pooled = x1[0:MCH]
    for k in range(1, K_AB):
        pooled = jnp.maximum(pooled, x1[k * MCH:(k + 1) * MCH])
    ph = jnp.dot(pooled, k2wa[...])
    phr = jnp.concatenate([ph] * K_AB, axis=0)
    x2 = _relu(phr + jnp.dot(x1, k2wb[...]) + k2b1[...])
    x2 = _relu(jnp.dot(x2, k2w2[...]) + k2b2[...])
    nbf = x2[0:MCH]
    for k in range(1, K_AB):
        nbf = jnp.maximum(nbf, x2[k * MCH:(k + 1) * MCH])
    nbf_ref[0] = nbf


def _partb1(cm, nbt, cmt, nafp, ws):
    wspecs = [pl.BlockSpec(w.shape, lambda b, mc: (0,) * w.ndim) for w in ws]
    nw_b = nafp.shape[1]
    return pl.pallas_call(
        _b1_body,
        grid=(B, MC),
        in_specs=[
            pl.BlockSpec((1, 3, MA), lambda b, mc: (b, 0, 0)),
            pl.BlockSpec((1, MCH, 3), lambda b, mc: (b, mc, 0)),
            pl.BlockSpec((1, MA, 3), lambda b, mc: (b, 0, 0)),
            pl.BlockSpec((1, nw_b, MA, 64), lambda b, mc: (b, 0, 0, 0)),
        ] + wspecs,
        out_specs=[
            pl.BlockSpec((1, MCH, 256), lambda b, mc: (b, mc, 0)),
            pl.BlockSpec((1, MA, 64), lambda b, mc: (b, 0, 0)),
        ],
        out_shape=[
            jax.ShapeDtypeStruct((B, MB, 256), jnp.float32),
            jax.ShapeDtypeStruct((B, MA, 64), jnp.float32),
        ],
    )(cm, nbt, cmt, nafp, *ws)


def _b2_body(nbt_ref, nbf_ref, w1a, w1b, b1, w2, b2, g_ref):
    x = _relu(jnp.dot(nbt_ref[0], w1a[...]) + jnp.dot(nbf_ref[0], w1b[...])
              + b1[...])
    x = _relu(jnp.dot(x, w2[...]) + b2[...])
    g_ref[0] = jnp.max(x, axis=0, keepdims=True)


def _partb2(nbt, nbf, ws):
    wspecs = [pl.BlockSpec(w.shape, lambda b: (0,) * w.ndim) for w in ws]
    return pl.pallas_call(
        _b2_body,
        grid=(B,),
        in_specs=[
            pl.BlockSpec((1, MB, 3), lambda b: (b, 0, 0)),
            pl.BlockSpec((1, MB, 256), lambda b: (b, 0, 0)),
        ] + wspecs,
        out_specs=[pl.BlockSpec((1, 1, 512), lambda b: (b, 0, 0))],
        out_shape=[jax.ShapeDtypeStruct((B, 1, 512), jnp.float32)],
    )(nbt, nbf, *ws)[0]


def kernel(pc, intensity, sn, label, node_a, node_b, params):
    p = params
    pct = pc.transpose(0, 2, 1)
    pcp = jnp.concatenate(
        [pct, jnp.ones((B, N, 1), jnp.float32)], axis=2)

    min3, cm = _pass1(pcp, node_a)
    mi = min3[:, :, :1]

    cmt = cm.transpose(0, 2, 1)
    cmp = jnp.concatenate(
        [cmt, jnp.zeros((B, MA, 5), jnp.float32)], axis=2)
    aug0 = jnp.concatenate(
        [pct, intensity.transpose(0, 2, 1), sn.transpose(0, 2, 1),
         label.transpose(0, 2, 1)], axis=2)

    fw = [w.T for w in p['fp_W']]
    fb = [b.reshape(1, -1) for b in p['fp_b']]
    ct, first = _pass2(aug0, mi, cmp, fw, fb)

    idx_bn = min3[:, :, 0]
    fmp = _segmax(first, idx_bn, 32)

    sw1 = p['sp_W'][0].T
    second = _pass3(first, mi, fmp, sw1[:32], sw1[32:],
                    p['sp_b'][0].reshape(1, -1),
                    p['sp_W'][1].T, p['sp_b'][1].reshape(1, -1))

    nafp = _segmax(second, idx_bn, 64)

    nbt = node_b.transpose(0, 2, 1)
    k1w1 = p['k1_W'][0].T
    k2w1 = p['k2_W'][0].T
    bws = [k1w1[:3], k1w1[3:], p['k1_b'][0].reshape(1, -1),
           p['k1_W'][1].T, p['k1_b'][1].reshape(1, -1),
           k2w1[:256], k2w1[256:], p['k2_b'][0].reshape(1, -1),
           p['k2_W'][1].T, p['k2_b'][1].reshape(1, -1)]
    nbf, naf = _partb1(cm, nbt, cmt, nafp, bws)

    fiw1 = p['fi_W'][0].T
    g = _partb2(nbt, nbf, [fiw1[:3], fiw1[3:], p['fi_b'][0].reshape(1, -1),
                           p['fi_W'][1].T, p['fi_b'][1].reshape(1, -1)])

    return (ct.transpose(0, 2, 1),
            cm,
            min3,
            first.transpose(0, 2, 1),
            second.transpose(0, 2, 1),
            naf.transpose(0, 2, 1),
            nbf.transpose(0, 2, 1),
            g.transpose(0, 2, 1))


# speedup vs baseline: 47.3631x; 1.0698x over previous
import functools

import jax
import jax.numpy as jnp
from jax import lax
from jax.experimental import pallas as pl
from jax.experimental.pallas import tpu as pltpu
from jax.experimental.pallas import tpu_sc as plsc

B, N, MA, MB = 2, 20480, 128, 128
K_AB = 32
TN = 4096
NT = N // TN
MCH = 64
MC = MB // MCH

_HI = jax.lax.Precision.HIGHEST


def _relu(x):
    return jnp.maximum(x, 0.0)


def _p1_body(pcp_ref, na_ref, min3_ref, cm_ref, sums_ref):
    nt = pl.program_id(1)
    pcp = pcp_ref[0]
    na = na_ref[0]
    d2 = None
    for c in range(3):
        t = pcp[:, c:c + 1] - na[c:c + 1, :]
        t2 = t * t
        d2 = t2 if d2 is None else d2 + t2
    d = jnp.sqrt(d2)
    lanes = lax.broadcasted_iota(jnp.int32, (TN, MA), 1).astype(jnp.float32)
    ik0 = None
    for k in range(3):
        mn = jnp.min(d, axis=1, keepdims=True)
        ik = jnp.min(jnp.where(d == mn, lanes, jnp.float32(MA)),
                     axis=1, keepdims=True)
        min3_ref[0, :, k:k + 1] = ik.astype(jnp.int32)
        if k == 0:
            ik0 = ik
        if k < 2:
            d = jnp.where(lanes == ik, jnp.float32(jnp.inf), d)
    mask = (lanes == ik0).astype(jnp.float32)
    part = lax.dot_general(pcp, mask, (((0,), (0,)), ((), ())),
                           preferred_element_type=jnp.float32,
                           precision=_HI)

    @pl.when(nt == 0)
    def _():
        sums_ref[...] = jnp.zeros_like(sums_ref)

    sums_ref[...] += part

    @pl.when(nt == NT - 1)
    def _():
        s = sums_ref[...]
        cm_ref[0] = s[:3] / (s[3:4] + 1e-5)


def _pass1(pcp, node_a):
    return pl.pallas_call(
        _p1_body,
        grid=(B, NT),
        in_specs=[
            pl.BlockSpec((1, TN, 4), lambda b, nt: (b, nt, 0)),
            pl.BlockSpec((1, 3, MA), lambda b, nt: (b, 0, 0)),
        ],
        out_specs=[
            pl.BlockSpec((1, TN, 3), lambda b, nt: (b, nt, 0)),
            pl.BlockSpec((1, 3, MA), lambda b, nt: (b, 0, 0)),
        ],
        out_shape=[
            jax.ShapeDtypeStruct((B, N, 3), jnp.int32),
            jax.ShapeDtypeStruct((B, 3, MA), jnp.float32),
        ],
        scratch_shapes=[pltpu.VMEM((4, MA), jnp.float32)],
    )(pcp, node_a)


def _p2_body(aug0_ref, mi_ref, cmp_ref, w1, b1, w2, b2, w3, b3,
             ct_ref, first_ref, firstT_ref):
    ik = mi_ref[0]
    lanes = lax.broadcasted_iota(jnp.int32, (TN, MA), 1)
    mask = (lanes == ik).astype(jnp.float32)
    centers8 = jnp.dot(mask, cmp_ref[0])
    ct_ref[0] = centers8[:, :3]
    x = aug0_ref[0] - centers8
    x = _relu(jnp.dot(x, w1[...]) + b1[...])
    x = _relu(jnp.dot(x, w2[...]) + b2[...])
    x = _relu(jnp.dot(x, w3[...]) + b3[...])
    first_ref[0] = x
    firstT_ref[0] = x.T


def _pass2(aug0, mi, cmp, fw, fb):
    wspecs = []
    for w in (fw[0], fb[0], fw[1], fb[1], fw[2], fb[2]):
        wspecs.append(pl.BlockSpec(w.shape, lambda b, nt: (0,) * w.ndim))
    return pl.pallas_call(
        _p2_body,
        grid=(B, NT),
        in_specs=[
            pl.BlockSpec((1, TN, 8), lambda b, nt: (b, nt, 0)),
            pl.BlockSpec((1, TN, 1), lambda b, nt: (b, nt, 0)),
            pl.BlockSpec((1, MA, 8), lambda b, nt: (b, 0, 0)),
        ] + wspecs,
        out_specs=[
            pl.BlockSpec((1, TN, 3), lambda b, nt: (b, nt, 0)),
            pl.BlockSpec((1, TN, 32), lambda b, nt: (b, nt, 0)),
            pl.BlockSpec((1, 32, TN), lambda b, nt: (b, 0, nt)),
        ],
        out_shape=[
            jax.ShapeDtypeStruct((B, N, 3), jnp.float32),
            jax.ShapeDtypeStruct((B, N, 32), jnp.float32),
            jax.ShapeDtypeStruct((B, 32, N), jnp.float32),
        ],
    )(aug0, mi, cmp, fw[0], fb[0], fw[1], fb[1], fw[2], fb[2])


def _segmax_sc(feat, idx, C):
    NC, NS = 2, 16
    NW = NC * NS
    n_per_w = feat.shape[1]
    G = C // 16
    mesh = plsc.VectorSubcoreMesh(core_axis_name="c", subcore_axis_name="s",
                                  num_cores=NC, num_subcores=NS)

    feat = feat.reshape(NW, n_per_w * C)

    @functools.partial(
        pl.kernel,
        out_type=jax.ShapeDtypeStruct((NW, MA * C), jnp.float32),
        mesh=mesh,
        scratch_types=[
            pltpu.VMEM((n_per_w * C,), jnp.float32),
            pltpu.VMEM((n_per_w,), jnp.int32),
            pltpu.VMEM((MA * C,), jnp.float32),
        ],
    )
    def k(feat_hbm, idx_hbm, out_hbm, feat_v, idx_v, acc_v):
        w = lax.axis_index("s") * NC + lax.axis_index("c")
        pltpu.sync_copy(feat_hbm.at[w], feat_v)
        pltpu.sync_copy(idx_hbm.at[w], idx_v)

        def zero(i, carry):
            acc_v[pl.ds(i * 16, 16)] = jnp.zeros((16,), jnp.float32)
            return carry

        lax.fori_loop(0, MA * C // 16, zero, 0)

        def body(i, carry):
            iv = idx_v[pl.ds(i * 16, 16)]
            for j in range(16):
                base = iv[j] * C
                fbase = (i * 16 + j) * C
                for g in range(G):
                    v = feat_v[pl.ds(fbase + g * 16, 16)]
                    a = acc_v[pl.ds(base + g * 16, 16)]
                    acc_v[pl.ds(base + g * 16, 16)] = jnp.maximum(a, v)
            return carry

        lax.fori_loop(0, n_per_w // 16, body, 0)
        pltpu.sync_copy(acc_v, out_hbm.at[w])

    return k(feat, idx)


def _segmax(feat_bnc, idx_bn, C):
    NW = 32
    n_per_w = (B * N) // NW
    f = feat_bnc.reshape(NW, n_per_w, C)
    i = idx_bn.reshape(NW, n_per_w)
    part = _segmax_sc(f, i, C)
    return part.reshape(B, NW // B, MA, C)


def _p3_body(first_ref, mi_ref, fmp_ref, w1a, w1b, b1, w2, b2,
             second_ref, secondT_ref):
    fm = fmp_ref[0, 0]
    for t in range(1, fmp_ref.shape[1]):
        fm = jnp.maximum(fm, fmp_ref[0, t])
    ik = mi_ref[0]
    lanes = lax.broadcasted_iota(jnp.int32, (TN, MA), 1)
    mask = (lanes == ik).astype(jnp.float32)
    sc = jnp.dot(mask, fm)
    f = first_ref[0]
    h = _relu(jnp.dot(f, w1a[...]) + jnp.dot(sc, w1b[...]) + b1[...])
    out = _relu(jnp.dot(h, w2[...]) + b2[...])
    second_ref[0] = out
    secondT_ref[0] = out.T


def _pass3(first, mi, fmp, w1a, w1b, b1, w2, b2):
    wspecs = [pl.BlockSpec(w.shape, lambda b, nt: (0,) * w.ndim)
              for w in (w1a, w1b, b1, w2, b2)]
    nw_b = fmp.shape[1]
    return pl.pallas_call(
        _p3_body,
        grid=(B, NT),
        in_specs=[
            pl.BlockSpec((1, TN, 32), lambda b, nt: (b, nt, 0)),
            pl.BlockSpec((1, TN, 1), lambda b, nt: (b, nt, 0)),
            pl.BlockSpec((1, nw_b, MA, 32), lambda b, nt: (b, 0, 0, 0)),
        ] + wspecs,
        out_specs=[
            pl.BlockSpec((1, TN, 64), lambda b, nt: (b, nt, 0)),
            pl.BlockSpec((1, 64, TN), lambda b, nt: (b, 0, nt)),
        ],
        out_shape=[
            jax.ShapeDtypeStruct((B, N, 64), jnp.float32),
            jax.ShapeDtypeStruct((B, 64, N), jnp.float32),
        ],
    )(first, mi, fmp, w1a, w1b, b1, w2, b2)


def _b1_body(cm_ref, nbt_ref, cmt_ref, nafp_ref,
             k1wa, k1wb, k1b1, k1w2, k1b2,
             k2wa, k2wb, k2b1, k2w2, k2b2,
             fiwa, fiwb, fib1, fiw2, fib2,
             nbf_ref, naf_ref, g_ref, gmax_ref):
    mc = pl.program_id(1)
    naf = nafp_ref[0, 0]
    for t in range(1, nafp_ref.shape[1]):
        naf = jnp.maximum(naf, nafp_ref[0, t])

    @pl.when(mc == 0)
    def _():
        naf_ref[0] = naf

    cm = cm_ref[0]
    nbc = nbt_ref[0]
    d2 = None
    for c in range(3):
        t = nbc[:, c:c + 1] - cm[c:c + 1, :]
        t2 = t * t
        d2 = t2 if d2 is None else d2 + t2
    d = jnp.sqrt(d2 + 1e-12)
    lanes = lax.broadcasted_iota(jnp.int32, (MCH, MA), 1).astype(jnp.float32)
    cols = []
    for k in range(K_AB):
        mn = jnp.min(d, axis=1, keepdims=True)
        ik = jnp.min(jnp.where(d == mn, lanes, jnp.float32(MA)),
                     axis=1, keepdims=True)
        cols.append(ik)
        if k < K_AB - 1:
            d = jnp.where(lanes == ik, jnp.float32(jnp.inf), d)
    idxcol = jnp.concatenate(cols, axis=0)
    R = K_AB * MCH
    lanesr = lax.broadcasted_iota(jnp.int32, (R, MA), 1).astype(jnp.float32)
    O = (lanesr == idxcol).astype(jnp.float32)
    nb_coord = jnp.dot(O, cmt_ref[0])
    nbb = jnp.concatenate([nbc] * K_AB, axis=0)
    rel = nb_coord - nbb
    nb_feat = jnp.dot(O, naf)
    x1 = _relu(jnp.dot(rel, k1wa[...]) + jnp.dot(nb_feat, k1wb[...])
               + k1b1[...])
    x1 = _relu(jnp.dot(x1, k1w2[...]) + k1b2[...])
    pooled = x1[0:MCH]
    for k in range(1, K_AB):
        pooled = jnp.maximum(pooled, x1[k * MCH:(k + 1) * MCH])
    ph = jnp.dot(pooled, k2wa[...])
    phr = jnp.concatenate([ph] * K_AB, axis=0)
    x2 = _relu(phr + jnp.dot(x1, k2wb[...]) + k2b1[...])
    x2 = _relu(jnp.dot(x2, k2w2[...]) + k2b2[...])
    nbf = x2[0:MCH]
    for k in range(1, K_AB):
        nbf = jnp.maximum(nbf, x2[k * MCH:(k + 1) * MCH])
    nbf_ref[0] = nbf
    fi = _relu(jnp.dot(nbc, fiwa[...]) + jnp.dot(nbf, fiwb[...]) + fib1[...])
    fi = _relu(jnp.dot(fi, fiw2[...]) + fib2[...])
    gm = jnp.max(fi, axis=0, keepdims=True)

    @pl.when(mc == 0)
    def _():
        gmax_ref[...] = jnp.zeros_like(gmax_ref)

    gmax_ref[...] = jnp.maximum(gmax_ref[...], gm)

    @pl.when(mc == MC - 1)
    def _():
        g_ref[0] = gmax_ref[...]


def _partb1(cm, nbt, cmt, nafp, ws):
    wspecs = [pl.BlockSpec(w.shape, lambda b, mc: (0,) * w.ndim) for w in ws]
    nw_b = nafp.shape[1]
    return pl.pallas_call(
        _b1_body,
        grid=(B, MC),
        in_specs=[
            pl.BlockSpec((1, 3, MA), lambda b, mc: (b, 0, 0)),
            pl.BlockSpec((1, MCH, 3), lambda b, mc: (b, mc, 0)),
            pl.BlockSpec((1, MA, 3), lambda b, mc: (b, 0, 0)),
            pl.BlockSpec((1, nw_b, MA, 64), lambda b, mc: (b, 0, 0, 0)),
        ] + wspecs,
        out_specs=[
            pl.BlockSpec((1, MCH, 256), lambda b, mc: (b, mc, 0)),
            pl.BlockSpec((1, MA, 64), lambda b, mc: (b, 0, 0)),
            pl.BlockSpec((1, 1, 512), lambda b, mc: (b, 0, 0)),
        ],
        out_shape=[
            jax.ShapeDtypeStruct((B, MB, 256), jnp.float32),
            jax.ShapeDtypeStruct((B, MA, 64), jnp.float32),
            jax.ShapeDtypeStruct((B, 1, 512), jnp.float32),
        ],
        scratch_shapes=[pltpu.VMEM((1, 512), jnp.float32)],
    )(cm, nbt, cmt, nafp, *ws)


def kernel(pc, intensity, sn, label, node_a, node_b, params):
    p = params
    pct = pc.transpose(0, 2, 1)
    pcp = jnp.concatenate(
        [pct, jnp.ones((B, N, 1), jnp.float32)], axis=2)

    min3, cm = _pass1(pcp, node_a)
    mi = min3[:, :, :1]

    cmt = cm.transpose(0, 2, 1)
    cmp = jnp.concatenate(
        [cmt, jnp.zeros((B, MA, 5), jnp.float32)], axis=2)
    aug0 = jnp.concatenate(
        [pct, intensity.transpose(0, 2, 1), sn.transpose(0, 2, 1),
         label.transpose(0, 2, 1)], axis=2)

    fw = [w.T for w in p['fp_W']]
    fb = [b.reshape(1, -1) for b in p['fp_b']]
    ct, first, first_cn = _pass2(aug0, mi, cmp, fw, fb)

    idx_bn = min3[:, :, 0]
    fmp = _segmax(first, idx_bn, 32)

    sw1 = p['sp_W'][0].T
    second, second_cn = _pass3(first, mi, fmp, sw1[:32], sw1[32:],
                               p['sp_b'][0].reshape(1, -1),
                               p['sp_W'][1].T, p['sp_b'][1].reshape(1, -1))

    nafp = _segmax(second, idx_bn, 64)

    nbt = node_b.transpose(0, 2, 1)
    k1w1 = p['k1_W'][0].T
    k2w1 = p['k2_W'][0].T
    bws = [k1w1[:3], k1w1[3:], p['k1_b'][0].reshape(1, -1),
           p['k1_W'][1].T, p['k1_b'][1].reshape(1, -1),
           k2w1[:256], k2w1[256:], p['k2_b'][0].reshape(1, -1),
           p['k2_W'][1].T, p['k2_b'][1].reshape(1, -1)]
    fiw1 = p['fi_W'][0].T
    bws += [fiw1[:3], fiw1[3:], p['fi_b'][0].reshape(1, -1),
            p['fi_W'][1].T, p['fi_b'][1].reshape(1, -1)]
    nbf, naf, g = _partb1(cm, nbt, cmt, nafp, bws)

    return (ct.transpose(0, 2, 1),
            cm,
            min3,
            first_cn,
            second_cn,
            naf.transpose(0, 2, 1),
            nbf.transpose(0, 2, 1),
            g.transpose(0, 2, 1))
